# baseline (speedup 1.0000x reference)
import functools

import jax
import jax.numpy as jnp
from jax import lax
from jax.experimental import pallas as pl
from jax.experimental.pallas import tpu as pltpu
from jax.experimental.pallas import tpu_sc as plsc

_NC = 2
_NS = 16
_L = 16
_NPAD = 128
_NB = 4


def _gather_kernel(N, HW, C, H, W):
    b_per_t = N // _NS
    n_chunks = b_per_t // 128
    cw = C // 2
    hw2 = cw // 2
    n_rows = HW + _NPAD
    rows_per_tile = n_rows // _NS

    mesh = plsc.VectorSubcoreMesh(core_axis_name="c", subcore_axis_name="s")

    @functools.partial(
        pl.kernel,
        mesh=mesh,
        compiler_params=pltpu.CompilerParams(use_tc_tiling_on_sc=False),
        out_type=jax.ShapeDtypeStruct((_NC, N // 128, 128, hw2), jnp.int32),
        scratch_types=(
            [pltpu.VMEM_SHARED((n_rows, hw2), jnp.int32)]
            + [pltpu.VMEM((128,), jnp.int32) for _ in range(3 * _NB)]
            + [pltpu.VMEM((128,), jnp.int32) for _ in range(_NB)]
            + [pltpu.VMEM((128, hw2), jnp.int32) for _ in range(_NB)]
            + [pltpu.SemaphoreType.DMA for _ in range(3 * _NB)]
        ),
    )
    def k(table_hbm, px_hbm, py_hbm, fov_hbm, out_hbm, table_sp, *scr):
        pxs = scr[:_NB]
        pys = scr[_NB:2 * _NB]
        fvs = scr[2 * _NB:3 * _NB]
        idxs = scr[3 * _NB:4 * _NB]
        bufs = scr[4 * _NB:5 * _NB]
        isems = scr[5 * _NB:6 * _NB]
        gsems = scr[6 * _NB:7 * _NB]
        wsems = scr[7 * _NB:8 * _NB]

        cid = lax.axis_index("c")
        sid = lax.axis_index("s")
        base = sid * b_per_t

        pltpu.sync_copy(
            table_hbm.at[cid, pl.ds(sid * rows_per_tile, rows_per_tile)],
            table_sp.at[pl.ds(sid * rows_per_tile, rows_per_tile)])
        plsc.subcore_barrier()

        lane = lax.broadcasted_iota(jnp.int32, (_L,), 0)

        def in_start(j, s):
            off = base + j * 128
            pltpu.async_copy(px_hbm.at[pl.ds(off, 128)], pxs[s], isems[s])
            pltpu.async_copy(py_hbm.at[pl.ds(off, 128)], pys[s], isems[s])
            pltpu.async_copy(fov_hbm.at[pl.ds(off, 128)], fvs[s], isems[s])

        def in_wait(j, s):
            off = base + j * 128
            pltpu.make_async_copy(
                px_hbm.at[pl.ds(off, 128)], pxs[s], isems[s]).wait()
            pltpu.make_async_copy(
                py_hbm.at[pl.ds(off, 128)], pys[s], isems[s]).wait()
            pltpu.make_async_copy(
                fov_hbm.at[pl.ds(off, 128)], fvs[s], isems[s]).wait()

        def idx_compute(s):
            pv, yv_, fv_, iv = pxs[s], pys[s], fvs[s], idxs[s]

            def body(v, _):
                xv = pv[pl.ds(v * _L, _L)]
                yv = yv_[pl.ds(v * _L, _L)]
                fv = fv_[pl.ds(v * _L, _L)]
                xc = jnp.clip(xv, 0, W - 1)
                yc = jnp.clip(yv, 0, H - 1)
                pad = HW + v * _L + lane
                iv[pl.ds(v * _L, _L)] = jnp.where(fv > 0, yc * W + xc, pad)
                return 0

            lax.fori_loop(0, 8, body, 0)

        def g_start(s):
            pltpu.async_copy(table_sp.at[idxs[s]], bufs[s], gsems[s])

        def g_wait(s):
            pltpu.make_async_copy(
                table_sp.at[idxs[s]], bufs[s], gsems[s]).wait()

        def w_start(j, s):
            pltpu.async_copy(
                bufs[s], out_hbm.at[cid, sid * n_chunks + j], wsems[s])

        def w_wait(j, s):
            pltpu.make_async_copy(
                bufs[s], out_hbm.at[cid, sid * n_chunks + j], wsems[s]).wait()

        def step(j, first, last):
            s = j % _NB
            s3 = (j + _NB - 1) % _NB
            g_wait(s)
            w_start(j, s)
            if not first:
                w_wait(j - 1, s3)
            if (not last) or j + _NB - 1 < n_chunks:
                g_start(s3)
            if (not last) or j + _NB < n_chunks:
                in_wait(j + _NB, s)
                idx_compute(s)
            if (not last) or j + 2 * _NB - 1 < n_chunks:
                in_start(j + 2 * _NB - 1, s3)

        for s in range(_NB):
            in_start(s, s)
        for c in range(_NB):
            in_wait(c, c)
            idx_compute(c)
            if c + _NB < 2 * _NB - 1:
                in_start(c + _NB, c)
        for c in range(_NB - 1):
            g_start(c)

        for j in range(8):
            step(j, j == 0, False)

        def outer(go, _):
            for u in range(8):
                step_j = go * 8 + u
                s = u % _NB
                s3 = (u + _NB - 1) % _NB
                g_wait(s)
                w_start(step_j, s)
                w_wait(step_j - 1, s3)
                g_start(s3)
                in_wait(step_j + _NB, s)
                idx_compute(s)
                in_start(step_j + 2 * _NB - 1, s3)
            return 0

        lax.fori_loop(1, n_chunks // 8 - 1, outer, 0)

        for j in range(n_chunks - 8, n_chunks):
            step(j, False, True)

        w_wait(n_chunks - 1, (n_chunks - 1) % _NB)

    return k


def kernel(x2d, projected_pix, fov_mask):
    bs, c, h, w = x2d.shape
    n = projected_pix.shape[1]
    hw = h * w

    table = jnp.concatenate(
        [x2d.reshape(c, hw).T, jnp.zeros((_NPAD, c), jnp.float32)],
        axis=0).astype(jnp.bfloat16)
    table = lax.bitcast_convert_type(
        table.reshape(hw + _NPAD, c // 2, 2), jnp.int32)
    table = table.reshape(hw + _NPAD, 2, c // 4).transpose(1, 0, 2)
    px = projected_pix[0, :, 0]
    py = projected_pix[0, :, 1]
    fov = fov_mask[0].astype(jnp.int32)

    y = _gather_kernel(n, hw, c, h, w)(table, px, py, fov)
    y = lax.bitcast_convert_type(y, jnp.bfloat16)
    y = y.reshape(2, n // 128, 128, c // 2).transpose(0, 3, 1, 2)

    sx, sy, sz = 128, 128, 16
    return y.reshape(c, n).astype(jnp.float32).reshape(bs, c, sx, sy, sz)


# speedup vs baseline: 4.0066x; 4.0066x over previous
import functools

import jax
import jax.numpy as jnp
from jax import lax
from jax.experimental import pallas as pl
from jax.experimental.pallas import tpu as pltpu
from jax.experimental.pallas import tpu_sc as plsc

_NC = 2
_NS = 16
_NW = _NC * _NS
_L = 16
_NPAD = 128
_NB = 5


def _gather_kernel(N, HW, C, H, W):
    b_per_w = N // _NW
    n_chunks = b_per_w // 128
    n_vec = b_per_w // _L

    mesh = plsc.VectorSubcoreMesh(core_axis_name="c", subcore_axis_name="s")

    @functools.partial(
        pl.kernel,
        mesh=mesh,
        out_type=jax.ShapeDtypeStruct((N, C), jnp.float32),
        scratch_types=[
            pltpu.VMEM((b_per_w,), jnp.int32),
            pltpu.VMEM((b_per_w,), jnp.int32),
            pltpu.VMEM((b_per_w,), jnp.int32),
            pltpu.VMEM((n_chunks, 128), jnp.int32),
        ]
        + [pltpu.VMEM((128, C), jnp.float32) for _ in range(_NB)]
        + [pltpu.SemaphoreType.DMA for _ in range(2 * _NB)],
    )
    def k(table_hbm, px_hbm, py_hbm, fov_hbm, out_hbm,
          px_v, py_v, fov_v, idx_v, *bufs_and_sems):
        bufs = bufs_and_sems[:_NB]
        gsems = bufs_and_sems[_NB:2 * _NB]
        wsems = bufs_and_sems[2 * _NB:]

        wid = lax.axis_index("s") * _NC + lax.axis_index("c")
        base = wid * b_per_w

        pltpu.sync_copy(px_hbm.at[pl.ds(base, b_per_w)], px_v)
        pltpu.sync_copy(py_hbm.at[pl.ds(base, b_per_w)], py_v)
        pltpu.sync_copy(fov_hbm.at[pl.ds(base, b_per_w)], fov_v)

        lane = lax.broadcasted_iota(jnp.int32, (_L,), 0)

        def idx_body(j, _):
            xv = px_v[pl.ds(j * _L, _L)]
            yv = py_v[pl.ds(j * _L, _L)]
            fv = fov_v[pl.ds(j * _L, _L)]
            xc = jnp.clip(xv, 0, W - 1)
            yc = jnp.clip(yv, 0, H - 1)
            pad = HW + (j % 8) * _L + lane
            idx = jnp.where(fv > 0, yc * W + xc, pad)
            idx_v[j // 8, pl.ds((j % 8) * _L, _L)] = idx
            return 0

        lax.fori_loop(0, n_vec, idx_body, 0, unroll=8)

        def g_start(j, s):
            pltpu.async_copy(table_hbm.at[idx_v.at[j]], bufs[s], gsems[s])

        def g_wait(j, s):
            pltpu.make_async_copy(
                table_hbm.at[idx_v.at[j]], bufs[s], gsems[s]).wait()

        def w_start(j, s):
            pltpu.async_copy(
                bufs[s], out_hbm.at[pl.ds(base + j * 128, 128)], wsems[s])

        def w_wait(j, s):
            pltpu.make_async_copy(
                bufs[s], out_hbm.at[pl.ds(base + j * 128, 128)],
                wsems[s]).wait()

        for j in range(_NB - 1):
            g_start(j, j)
        for g in range(n_chunks):
            s = g % _NB
            g_wait(g, s)
            w_start(g, s)
            jn = g + _NB - 1
            if jn < n_chunks:
                s2 = jn % _NB
                if g >= 1:
                    w_wait(g - 1, s2)
                g_start(jn, s2)
        for j in range(n_chunks - _NB, n_chunks):
            w_wait(j, j % _NB)

    return k


def kernel(x2d, projected_pix, fov_mask):
    bs, c, h, w = x2d.shape
    n = projected_pix.shape[1]
    hw = h * w

    table = jnp.concatenate(
        [x2d.reshape(c, hw).T, jnp.zeros((_NPAD, c), jnp.float32)], axis=0)
    px = projected_pix[0, :, 0]
    py = projected_pix[0, :, 1]
    fov = fov_mask[0].astype(jnp.int32)

    y = _gather_kernel(n, hw, c, h, w)(table, px, py, fov)

    sx, sy, sz = 128, 128, 16
    return y.T.reshape(bs, c, sx, sy, sz)
